# baseline (speedup 1.0000x reference)
import jax
import jax.numpy as jnp
from jax.experimental import pallas as pl
from jax.experimental.pallas import tpu as pltpu

_TB = 32
_K = 4


def _log_softmax(x):
    m = jnp.max(x, axis=-1, keepdims=True)
    s = x - m
    return s - jnp.log(jnp.sum(jnp.exp(s), axis=-1, keepdims=True))


def _prep_kernel(tn_ref, pn_ref, tn_out_ref, pn_out_ref):
    tn_out_ref[...] = _log_softmax(tn_ref[...])
    pn = _log_softmax(pn_ref[...])
    v = jax.lax.broadcasted_iota(jnp.int32, pn.shape, 1)
    pn_out_ref[...] = jnp.where(v == 0, 0.0, pn)


def _add_kernel(tn_ref, pn_ref, out_ref, scratch, sems):
    b = pl.program_id(0)
    t = pl.program_id(1)
    nt = pl.num_programs(1)
    step = b * nt + t
    slot = jax.lax.rem(step, _K)
    tb = tn_ref.shape[0]

    def copy_for(s):
        return pltpu.make_async_copy(
            scratch.at[s],
            out_ref.at[b, pl.ds(t * tb, tb), :, :],
            sems.at[s],
        )

    @pl.when(step >= _K)
    def _():
        copy_for(slot).wait()

    scratch[slot] = tn_ref[...][:, None, :] + pn_ref[...][None, :, :]
    copy_for(slot).start()

    @pl.when(step == pl.num_programs(0) * nt - 1)
    def _():
        for s in range(_K):
            copy_for(s).wait()


def kernel(tn_out, pn_out):
    B, T, V = tn_out.shape
    _, U, _ = pn_out.shape
    tn_ls, pn_ls = pl.pallas_call(
        _prep_kernel,
        grid=(B,),
        in_specs=[
            pl.BlockSpec((None, T, V), lambda b: (b, 0, 0)),
            pl.BlockSpec((None, U, V), lambda b: (b, 0, 0)),
        ],
        out_specs=[
            pl.BlockSpec((None, T, V), lambda b: (b, 0, 0)),
            pl.BlockSpec((None, U, V), lambda b: (b, 0, 0)),
        ],
        out_shape=[
            jax.ShapeDtypeStruct((B, T, V), tn_out.dtype),
            jax.ShapeDtypeStruct((B, U, V), pn_out.dtype),
        ],
    )(tn_out, pn_out)

    Tb = _TB
    return pl.pallas_call(
        _add_kernel,
        grid=(B, T // Tb),
        in_specs=[
            pl.BlockSpec((None, Tb, V), lambda b, t: (b, t, 0)),
            pl.BlockSpec((None, U, V), lambda b, t: (b, 0, 0)),
        ],
        out_specs=pl.BlockSpec(memory_space=pl.ANY),
        out_shape=jax.ShapeDtypeStruct((B, T, U, V), tn_out.dtype),
        scratch_shapes=[
            pltpu.VMEM((_K, Tb, U, V), tn_out.dtype),
            pltpu.SemaphoreType.DMA((_K,)),
        ],
    )(tn_ls, pn_ls)
